# baseline (speedup 1.0000x reference)
import functools

import jax
import jax.numpy as jnp
from jax import lax
from jax.experimental import pallas as pl
from jax.experimental.pallas import tpu as pltpu
from jax.experimental.pallas import tpu_sc as plsc

BIG = 2**30


def _tc_body(n_nodes, n_samples, s_pad, x_ref, px_ref, py_ref, pz_ref,
             posr_ref, iflatf_ref, wt_ref, b_ref, gamma_ref, beta_ref,
             h_ref, idx_ref):
    h_lin = jnp.dot(x_ref[...], wt_ref[...],
                    preferred_element_type=jnp.float32) + b_ref[...]
    mean = jnp.sum(h_lin, axis=0, keepdims=True) / float(n_nodes)
    cen = h_lin - mean
    var = jnp.sum(cen * cen, axis=0, keepdims=True) / float(n_nodes)
    h = cen / jnp.sqrt(var + 1e-5) * gamma_ref[...] + beta_ref[...]
    h_ref[...] = jnp.maximum(h, 0.0)

    idx_ref[...] = jnp.zeros((s_pad, 1), jnp.int32)
    px = px_ref[...]
    py = py_ref[...]
    pz = pz_ref[...]
    valid = iflatf_ref[...] < float(n_nodes)

    q0x = posr_ref[0]
    q0y = posr_ref[1]
    q0z = posr_ref[2]
    dx = px - q0x
    dy = py - q0y
    dz = pz - q0z
    dist0 = (dx * dx + dy * dy) + dz * dz
    dist0 = jnp.where(valid, dist0, -1.0)

    def fps_body(it, dist):
        m = jnp.max(dist, axis=(0, 1), keepdims=True)
        flatf = jnp.min(jnp.where(dist == m, iflatf_ref[...], 3e9))
        flat = flatf.astype(jnp.int32)
        idx_ref[pl.ds(it, 1), :] = jnp.reshape(flat, (1, 1))
        f3 = flat * 3
        qx = posr_ref[f3]
        qy = posr_ref[f3 + 1]
        qz = posr_ref[f3 + 2]
        ddx = px_ref[...] - qx
        ddy = py_ref[...] - qy
        ddz = pz_ref[...] - qz
        d = (ddx * ddx + ddy * ddy) + ddz * ddz
        return jnp.minimum(dist, d)

    lax.fori_loop(1, n_samples, fps_body, dist0)


def _tc_stage(x, pos, wt, b, gamma, beta, n_samples, s_pad):
    n, d = x.shape
    nrow = (n + 127) // 128
    npad = nrow * 128
    pos_t = jnp.transpose(pos)
    pos_t = jnp.pad(pos_t, ((0, 0), (0, npad - n)),
                    constant_values=1e9).reshape(3, nrow, 128)
    iflatf = jnp.arange(npad, dtype=jnp.float32).reshape(nrow, 128)
    body = functools.partial(_tc_body, n, n_samples, s_pad)
    vspec = pl.BlockSpec(memory_space=pltpu.VMEM)
    return pl.pallas_call(
        body,
        in_specs=[vspec, vspec, vspec, vspec,
                  pl.BlockSpec(memory_space=pltpu.SMEM),
                  vspec, vspec, vspec, vspec, vspec],
        out_shape=[jax.ShapeDtypeStruct((n, d), jnp.float32),
                   jax.ShapeDtypeStruct((s_pad, 1), jnp.int32)],
    )(x, pos_t[0], pos_t[1], pos_t[2], pos.reshape(-1), iflatf, wt,
      b.reshape(1, d), gamma.reshape(1, d), beta.reshape(1, d))


_NS = 16
_RPW = 320
_SPAD = _NS * _RPW
_CE = 6400
_GB = 128


def _sc_body(n_nodes, n_pad, n_samples, s_pad, d, e_half,
             h_hbm, idx_hbm, src_hbm, dst_hbm, pos4_hbm,
             outh0_hbm, outh1_hbm, outp_hbm,
             rank_ts, idxst_ts, idx_ts, acc_ts, dstbuf, srcbuf,
             msrc, mrank, rows_ts, rowpb_ts, posout_ts,
             sem):
    core = lax.axis_index("c")
    wid = lax.axis_index("s")
    lo = wid * _RPW
    hi = lo + _RPW
    ebase = core * e_half
    dseg = d // 16
    ones16 = jnp.ones((16,), jnp.int32)
    zeros16 = jnp.zeros((16,), jnp.int32)
    lane = lax.iota(jnp.int32, 16)

    def zero_mask(i, c):
        rank_ts[pl.ds(i * 16, 16)] = zeros16
        return c

    lax.fori_loop(0, n_pad // 16, zero_mask, 0)
    pltpu.sync_copy(idx_hbm, idxst_ts)

    def scatter_mask(i, c):
        v = idxst_ts[pl.ds(i * 16, 16)]
        m = (i * 16 + lane) < n_samples
        plsc.store_scatter(rank_ts, [v], ones16, mask=m)
        return c

    lax.fori_loop(0, s_pad // 16, scatter_mask, 0)

    def rank_vec(i, carry):
        mv = rank_ts[pl.ds(i * 16, 16)]
        incl = plsc.cumsum(mv) + carry
        rank_ts[pl.ds(i * 16, 16)] = jnp.where(mv > 0, incl - 1, BIG)
        return carry + jnp.sum(mv)

    lax.fori_loop(0, n_pad // 16, rank_vec, jnp.int32(0))

    def zero_idx(i, c):
        idx_ts[pl.ds(i * 16, 16)] = zeros16
        return c

    lax.fori_loop(0, _RPW // 16, zero_idx, 0)

    def scan_ids(i, c):
        rv = rank_ts[pl.ds(i * 16, 16)]
        ids = jnp.full((16,), i * 16, jnp.int32) + lane
        m = (rv >= lo) & (rv < hi)
        plsc.store_scatter(idx_ts, [rv - lo], ids, mask=m)
        return c

    lax.fori_loop(0, n_pad // 16, scan_ids, 0)

    for g in range(_RPW // 80):
        pltpu.async_copy(h_hbm.at[idx_ts.at[pl.ds(g * 80, 80)]],
                         acc_ts.at[pl.ds(g * 80, 80)], sem).wait()

    def zero_msrc(i, c):
        msrc[pl.ds(i * 16, 16)] = zeros16
        return c

    lax.fori_loop(0, _CE // 16, zero_msrc, 0)

    def chunk_body(ch, carry):
        base_e = ebase + ch * _CE
        pltpu.sync_copy(dst_hbm.at[pl.ds(base_e, _CE)], dstbuf)
        pltpu.sync_copy(src_hbm.at[pl.ds(base_e, _CE)], srcbuf)

        def scan_vec(i, p):
            for u in range(4):
                off = i * 64 + u * 16
                dstv = dstbuf[pl.ds(off, 16)]
                rv = plsc.load_gather(rank_ts, [dstv])
                m = (rv >= lo) & (rv < hi)
                srcv = srcbuf[pl.ds(off, 16)]
                cnt = plsc.all_reduce_population_count(m)[0]
                plsc.store_compressed(msrc.at[pl.ds(p, 16)], srcv, mask=m)
                plsc.store_compressed(mrank.at[pl.ds(p, 16)], rv, mask=m)
                p = p + cnt
            return p

        p = lax.fori_loop(0, _CE // 64, scan_vec, jnp.int32(0))

        def batch_cond(bi):
            return bi * _GB < p

        def batch_body(bi):
            gbase = bi * _GB
            pltpu.async_copy(h_hbm.at[msrc.at[pl.ds(gbase, _GB)]],
                             rows_ts, sem).wait()
            nvalid = jnp.minimum(_GB, p - gbase)

            def row_body(e, c):
                lr = mrank[pl.ds(gbase + e, 16)][0] - lo
                new = [rows_ts[e, pl.ds(j * 16, 16)] for j in range(dseg)]
                cur = [acc_ts[lr, pl.ds(j * 16, 16)] for j in range(dseg)]
                for j in range(dseg):
                    acc_ts[lr, pl.ds(j * 16, 16)] = jnp.maximum(cur[j], new[j])
                return c

            lax.fori_loop(0, nvalid, row_body, 0)
            return bi + 1

        lax.while_loop(batch_cond, batch_body, jnp.int32(0))
        return carry

    lax.fori_loop(0, e_half // _CE, chunk_body, 0)

    @pl.when(core == 0)
    def _():
        for g in range(_RPW // 80):
            pltpu.async_copy(pos4_hbm.at[idx_ts.at[pl.ds(g * 80, 80)]],
                             rowpb_ts, sem).wait()

            def pb_row(r, c):
                posout_ts[pl.ds((g * 80 + r) * 16, 16)] = \
                    rowpb_ts[r, pl.ds(0, 16)]
                return c

            lax.fori_loop(0, 80, pb_row, 0)
        pltpu.sync_copy(posout_ts, outp_hbm.at[pl.ds(lo * 16, _RPW * 16)])
        pltpu.sync_copy(acc_ts, outh0_hbm.at[pl.ds(lo, _RPW)])

    @pl.when(core == 1)
    def _():
        pltpu.sync_copy(acc_ts, outh1_hbm.at[pl.ds(lo, _RPW)])


def _sc_stage(h, idx_flat, src, dst, pos, batch, n_samples, n_pad):
    n, d = h.shape
    e = src.shape[0]
    mesh = plsc.VectorSubcoreMesh(core_axis_name="c", subcore_axis_name="s",
                                  num_cores=2, num_subcores=16)
    body = functools.partial(_sc_body, n, n_pad, n_samples, _SPAD, d, e // 2)
    kern = pl.kernel(
        body,
        out_type=[jax.ShapeDtypeStruct((_SPAD, d), jnp.float32),
                  jax.ShapeDtypeStruct((_SPAD, d), jnp.float32),
                  jax.ShapeDtypeStruct((_SPAD * 16,), jnp.float32)],
        mesh=mesh,
        compiler_params=pltpu.CompilerParams(needs_layout_passes=False),
        scratch_types=[
            pltpu.VMEM((n_pad,), jnp.int32),
            pltpu.VMEM((_SPAD,), jnp.int32),
            pltpu.VMEM((_RPW,), jnp.int32),
            pltpu.VMEM((_RPW, d), jnp.float32),
            pltpu.VMEM((_CE,), jnp.int32),
            pltpu.VMEM((_CE,), jnp.int32),
            pltpu.VMEM((_CE + 16,), jnp.int32),
            pltpu.VMEM((_CE + 16,), jnp.int32),
            pltpu.VMEM((_GB, d), jnp.float32),
            pltpu.VMEM((80, 128), jnp.float32),
            pltpu.VMEM((_RPW * 16,), jnp.float32),
            pltpu.SemaphoreType.DMA,
        ],
    )
    pos4 = jnp.concatenate(
        [pos, lax.bitcast_convert_type(batch, jnp.float32)[:, None]], axis=1)
    pos4 = jnp.pad(pos4, ((0, 0), (0, 124)))
    return kern(h, idx_flat, src, dst, pos4)


def _merge_body(a_ref, b_ref, o_ref):
    o_ref[...] = jnp.maximum(a_ref[...], b_ref[...])


def _merge_stage(a, b):
    return pl.pallas_call(
        _merge_body,
        out_shape=jax.ShapeDtypeStruct(a.shape, a.dtype),
    )(a, b)


def kernel(x, pos, edge_index, batch, W, b, gamma, beta):
    n = x.shape[0]
    n_samples = n // 2
    n_pad = ((n + 127) // 128) * 128
    h, idx2d = _tc_stage(x, pos, jnp.transpose(W), b, gamma, beta,
                         n_samples, _SPAD)
    outh0, outh1, outpb = _sc_stage(h, idx2d.reshape(-1), edge_index[0],
                                    edge_index[1], pos, batch,
                                    n_samples, n_pad)
    outh = _merge_stage(outh0, outh1)
    pb = outpb.reshape(_SPAD, 16)
    return (outh[:n_samples],
            pb[:n_samples, :3],
            lax.bitcast_convert_type(pb[:n_samples, 3], jnp.int32))


# speedup vs baseline: 1.2040x; 1.2040x over previous
import functools

import jax
import jax.numpy as jnp
from jax import lax
from jax.experimental import pallas as pl
from jax.experimental.pallas import tpu as pltpu
from jax.experimental.pallas import tpu_sc as plsc

BIG = 2**30


def _tc_body(n_nodes, n_samples, s_pad, x_ref, px_ref, py_ref, pz_ref,
             posr_ref, iflatf_ref, wt_ref, b_ref, gamma_ref, beta_ref,
             h_ref, idx_ref):
    h_lin = jnp.dot(x_ref[...], wt_ref[...],
                    preferred_element_type=jnp.float32) + b_ref[...]
    mean = jnp.sum(h_lin, axis=0, keepdims=True) / float(n_nodes)
    cen = h_lin - mean
    var = jnp.sum(cen * cen, axis=0, keepdims=True) / float(n_nodes)
    h = cen / jnp.sqrt(var + 1e-5) * gamma_ref[...] + beta_ref[...]
    h_ref[...] = jnp.maximum(h, 0.0)

    idx_ref[...] = jnp.zeros((s_pad, 1), jnp.int32)
    px = px_ref[...]
    py = py_ref[...]
    pz = pz_ref[...]
    valid = iflatf_ref[...] < float(n_nodes)

    q0x = posr_ref[0]
    q0y = posr_ref[1]
    q0z = posr_ref[2]
    dx = px - q0x
    dy = py - q0y
    dz = pz - q0z
    dist0 = (dx * dx + dy * dy) + dz * dz
    dist0 = jnp.where(valid, dist0, -1.0)

    def fps_body(it, dist):
        m = jnp.max(dist, axis=(0, 1), keepdims=True)
        flatf = jnp.min(jnp.where(dist == m, iflatf_ref[...], 3e9))
        flat = flatf.astype(jnp.int32)
        idx_ref[pl.ds(it, 1), :] = jnp.reshape(flat, (1, 1))
        f3 = flat * 3
        qx = posr_ref[f3]
        qy = posr_ref[f3 + 1]
        qz = posr_ref[f3 + 2]
        ddx = px_ref[...] - qx
        ddy = py_ref[...] - qy
        ddz = pz_ref[...] - qz
        d = (ddx * ddx + ddy * ddy) + ddz * ddz
        return jnp.minimum(dist, d)

    lax.fori_loop(1, n_samples, fps_body, dist0)


def _tc_stage(x, pos, wt, b, gamma, beta, n_samples, s_pad):
    n, d = x.shape
    nrow = (n + 127) // 128
    npad = nrow * 128
    pos_t = jnp.transpose(pos)
    pos_t = jnp.pad(pos_t, ((0, 0), (0, npad - n)),
                    constant_values=1e9).reshape(3, nrow, 128)
    iflatf = jnp.arange(npad, dtype=jnp.float32).reshape(nrow, 128)
    body = functools.partial(_tc_body, n, n_samples, s_pad)
    vspec = pl.BlockSpec(memory_space=pltpu.VMEM)
    return pl.pallas_call(
        body,
        in_specs=[vspec, vspec, vspec, vspec,
                  pl.BlockSpec(memory_space=pltpu.SMEM),
                  vspec, vspec, vspec, vspec, vspec],
        out_shape=[jax.ShapeDtypeStruct((n, d), jnp.float32),
                   jax.ShapeDtypeStruct((s_pad, 1), jnp.int32)],
    )(x, pos_t[0], pos_t[1], pos_t[2], pos.reshape(-1), iflatf, wt,
      b.reshape(1, d), gamma.reshape(1, d), beta.reshape(1, d))


_NS = 16
_RPW = 320
_SPAD = _NS * _RPW
_CE = 3200
_GB = 64


def _sc_body(n_nodes, n_pad, n_samples, s_pad, d, e_half,
             h_hbm, idx_hbm, src_hbm, dst_hbm, pos4_hbm,
             outh0_hbm, outh1_hbm, outp_hbm,
             rank_ts, idxst_ts, idx_ts, acc_ts, dstbuf, srcbuf,
             msrc, mrank, rows_ts, rowpb_ts, posout_ts,
             sem):
    core = lax.axis_index("c")
    wid = lax.axis_index("s")
    lo = wid * _RPW
    hi = lo + _RPW
    ebase = core * e_half
    dseg = d // 16
    ones16 = jnp.ones((16,), jnp.int32)
    zeros16 = jnp.zeros((16,), jnp.int32)
    lane = lax.iota(jnp.int32, 16)

    def zero_mask(i, c):
        rank_ts[pl.ds(i * 16, 16)] = zeros16
        return c

    lax.fori_loop(0, n_pad // 16, zero_mask, 0)
    pltpu.sync_copy(idx_hbm, idxst_ts)

    def scatter_mask(i, c):
        v = idxst_ts[pl.ds(i * 16, 16)]
        m = (i * 16 + lane) < n_samples
        plsc.store_scatter(rank_ts, [v], ones16, mask=m)
        return c

    lax.fori_loop(0, s_pad // 16, scatter_mask, 0)

    def rank_vec(i, carry):
        mv = rank_ts[pl.ds(i * 16, 16)]
        incl = plsc.cumsum(mv) + carry
        rank_ts[pl.ds(i * 16, 16)] = jnp.where(mv > 0, incl - 1, BIG)
        return carry + jnp.sum(mv)

    lax.fori_loop(0, n_pad // 16, rank_vec, jnp.int32(0))

    def zero_idx(i, c):
        idx_ts[pl.ds(i * 16, 16)] = zeros16
        return c

    lax.fori_loop(0, _RPW // 16, zero_idx, 0)

    def scan_ids(i, c):
        rv = rank_ts[pl.ds(i * 16, 16)]
        ids = jnp.full((16,), i * 16, jnp.int32) + lane
        m = (rv >= lo) & (rv < hi)
        plsc.store_scatter(idx_ts, [rv - lo], ids, mask=m)
        return c

    lax.fori_loop(0, n_pad // 16, scan_ids, 0)

    for g in range(_RPW // 80):
        pltpu.async_copy(h_hbm.at[idx_ts.at[pl.ds(g * 80, 80)]],
                         acc_ts.at[pl.ds(g * 80, 80)], sem).wait()

    def zero_msrc(i, c):
        msrc[pl.ds(i * 16, 16)] = zeros16
        return c

    lax.fori_loop(0, _CE // 16, zero_msrc, 0)

    def chunk_body(ch, carry):
        base_e = ebase + ch * _CE
        pltpu.sync_copy(dst_hbm.at[pl.ds(base_e, _CE)], dstbuf)
        pltpu.sync_copy(src_hbm.at[pl.ds(base_e, _CE)], srcbuf)

        def scan_vec(i, p):
            for u in range(4):
                off = i * 64 + u * 16
                dstv = dstbuf[pl.ds(off, 16)]
                rv = plsc.load_gather(rank_ts, [dstv])
                m = (rv >= lo) & (rv < hi)
                srcv = srcbuf[pl.ds(off, 16)]
                cnt = plsc.all_reduce_population_count(m)[0]
                plsc.store_compressed(msrc.at[pl.ds(p, 16)], srcv, mask=m)
                plsc.store_compressed(mrank.at[pl.ds(p, 16)], rv, mask=m)
                p = p + cnt
            return p

        p = lax.fori_loop(0, _CE // 64, scan_vec, jnp.int32(0))

        def batch_cond(bi):
            return bi * _GB < p

        def batch_body(bi):
            gbase = bi * _GB
            pltpu.async_copy(h_hbm.at[msrc.at[pl.ds(gbase, _GB)]],
                             rows_ts, sem).wait()
            nvalid = jnp.minimum(_GB, p - gbase)

            def row_body(e, c):
                lr = mrank[pl.ds(gbase + e, 16)][0] - lo
                new = [rows_ts[e, pl.ds(j * 16, 16)] for j in range(dseg)]
                cur = [acc_ts[lr, pl.ds(j * 16, 16)] for j in range(dseg)]
                for j in range(dseg):
                    acc_ts[lr, pl.ds(j * 16, 16)] = jnp.maximum(cur[j], new[j])
                return c

            lax.fori_loop(0, nvalid, row_body, 0)
            return bi + 1

        lax.while_loop(batch_cond, batch_body, jnp.int32(0))
        return carry

    lax.fori_loop(0, e_half // _CE, chunk_body, 0)

    @pl.when(core == 0)
    def _():
        for g in range(_RPW // 80):
            pltpu.async_copy(pos4_hbm.at[idx_ts.at[pl.ds(g * 80, 80)]],
                             rowpb_ts, sem).wait()

            def pb_row(r, c):
                posout_ts[pl.ds((g * 80 + r) * 16, 16)] = \
                    rowpb_ts[r, pl.ds(0, 16)]
                return c

            lax.fori_loop(0, 80, pb_row, 0)
        pltpu.sync_copy(posout_ts, outp_hbm.at[pl.ds(lo * 16, _RPW * 16)])
        pltpu.sync_copy(acc_ts, outh0_hbm.at[pl.ds(lo, _RPW)])

    @pl.when(core == 1)
    def _():
        pltpu.sync_copy(acc_ts, outh1_hbm.at[pl.ds(lo, _RPW)])


def _sc_stage(h, idx_flat, src, dst, pos, batch, n_samples, n_pad):
    n, d = h.shape
    e = src.shape[0]
    mesh = plsc.VectorSubcoreMesh(core_axis_name="c", subcore_axis_name="s",
                                  num_cores=2, num_subcores=16)
    body = functools.partial(_sc_body, n, n_pad, n_samples, _SPAD, d, e // 2)
    kern = pl.kernel(
        body,
        out_type=[jax.ShapeDtypeStruct((_SPAD, d), jnp.float32),
                  jax.ShapeDtypeStruct((_SPAD, d), jnp.float32),
                  jax.ShapeDtypeStruct((_SPAD * 16,), jnp.float32)],
        mesh=mesh,
        compiler_params=pltpu.CompilerParams(needs_layout_passes=False),
        scratch_types=[
            pltpu.VMEM((n_pad,), jnp.int32),
            pltpu.VMEM((_SPAD,), jnp.int32),
            pltpu.VMEM((_RPW,), jnp.int32),
            pltpu.VMEM((_RPW, d), jnp.float32),
            pltpu.VMEM((_CE,), jnp.int32),
            pltpu.VMEM((_CE,), jnp.int32),
            pltpu.VMEM((_CE + 16,), jnp.int32),
            pltpu.VMEM((_CE + 16,), jnp.int32),
            pltpu.VMEM((_GB, d), jnp.float32),
            pltpu.VMEM((80, 128), jnp.float32),
            pltpu.VMEM((_RPW * 16,), jnp.float32),
            pltpu.SemaphoreType.DMA,
        ],
    )
    pos4 = jnp.concatenate(
        [pos, lax.bitcast_convert_type(batch, jnp.float32)[:, None]], axis=1)
    pos4 = jnp.pad(pos4, ((0, 0), (0, 124)))
    return kern(h, idx_flat, src, dst, pos4)


def _merge_body(a_ref, b_ref, o_ref):
    o_ref[...] = jnp.maximum(a_ref[...], b_ref[...])


def _merge_stage(a, b):
    return pl.pallas_call(
        _merge_body,
        out_shape=jax.ShapeDtypeStruct(a.shape, a.dtype),
    )(a, b)


def kernel(x, pos, edge_index, batch, W, b, gamma, beta):
    n = x.shape[0]
    n_samples = n // 2
    n_pad = ((n + 127) // 128) * 128
    h, idx2d = _tc_stage(x, pos, jnp.transpose(W), b, gamma, beta,
                         n_samples, _SPAD)
    outh0, outh1, outpb = _sc_stage(h, idx2d.reshape(-1), edge_index[0],
                                    edge_index[1], pos, batch,
                                    n_samples, n_pad)
    outh = _merge_stage(outh0, outh1)
    pb = outpb.reshape(_SPAD, 16)
    return (outh[:n_samples],
            pb[:n_samples, :3],
            lax.bitcast_convert_type(pb[:n_samples, 3], jnp.int32))
